# Initial kernel scaffold; baseline (speedup 1.0000x reference)
import jax
import jax.numpy as jnp
from jax.experimental import pallas as pl


def kernel(wid, edge_index, tree_vec, p_targets, q_targets, emb, Wz, bz, Wr, Ur, bUr, Wh, bh, W_W, b_W, W_o, b_o, W_U, b_U, U_s, b_s):
    raise NotImplementedError("write your pallas kernel here")


# speedup vs baseline: 5.2540x; 5.2540x over previous
import functools

import jax
import jax.numpy as jnp
from jax import lax
from jax.experimental import pallas as pl
from jax.experimental.pallas import tpu as pltpu
from jax.experimental.pallas import tpu_sc as plsc

N = 10000
E = 320000
H = 128
L = 56
V = 780
NT = 256.0

NP = 10240
VP = 784
EROWS = 2560
NA = 10112
RPT = NA // 16
DUMMY = N

_f32 = jnp.float32


def _tables_body(emb, wz1, wh1, wr, ur, wu1, bz, bh, bur,
                 m1t_o, rmt_o, ewz1_o, ewh1_o, ewu1_o):
    e = emb[...]
    ewz1 = jnp.dot(e, wz1[...], preferred_element_type=_f32)
    ewh1 = jnp.dot(e, wh1[...], preferred_element_type=_f32)
    m1 = jax.nn.sigmoid(ewz1 + bz[...]) * jnp.tanh(ewh1 + bh[...])
    rt = jax.nn.sigmoid(jnp.dot(e, wr[...], preferred_element_type=_f32)
                        + jnp.dot(m1, ur[...], preferred_element_type=_f32)
                        + bur[...])
    m1t_o[...] = m1
    rmt_o[...] = rt * m1
    ewz1_o[...] = ewz1
    ewh1_o[...] = ewh1
    ewu1_o[...] = jnp.dot(e, wu1[...], preferred_element_type=_f32)


_tables_tc = pl.pallas_call(
    _tables_body,
    out_shape=[jax.ShapeDtypeStruct((VP, H), _f32) for _ in range(5)],
)


def _seg_accum(src2, dst2, tbl, accum, src_v, dst_v, rows_v, sems,
               row_base, nslabs):
    def body(i, carry):
        r0 = row_base + 8 * i
        pltpu.sync_copy(src2.at[pl.ds(r0, 8)], src_v)
        pltpu.sync_copy(dst2.at[pl.ds(r0, 8)], dst_v)
        cp = pltpu.async_copy(tbl.at[src_v.at[0]],
                              rows_v.at[pl.ds(0, 128)], sems[0])
        for j in range(8):
            nxt = None
            if j < 7:
                nxt = pltpu.async_copy(tbl.at[src_v.at[j + 1]],
                                       rows_v.at[pl.ds(((j + 1) % 2) * 128,
                                                       128)],
                                       sems[(j + 1) % 2])
            cp.wait()
            pltpu.sync_copy(rows_v.at[pl.ds((j % 2) * 128, 128)],
                            accum.at[dst_v.at[j]], add=True)
            cp = nxt
        return carry

    lax.fori_loop(0, nslabs, body, 0)


def _gather_rows(tbl, out, widx_row, rows_v, sem, out_row):
    pltpu.async_copy(tbl.at[widx_row], rows_v.at[pl.ds(0, 128)], sem).wait()
    pltpu.sync_copy(rows_v.at[pl.ds(0, 128)], out.at[pl.ds(out_row * 128, 128)])


def _sc1_body(wid2d, src2, dst2, m1t, rmt, ewz1, ewh1, ewu1, zrows,
              s2_o, a2_o, m1n_o, rmn_o, xwz1_o, xwh1_o, xwu1_o,
              widx_v, src_v, dst_v, rows_v, accum, semA, semB):
    sem = semA
    sems = (semA, semB)
    c = lax.axis_index("c")
    s = lax.axis_index("s")
    gw = s * 2 + c

    pltpu.sync_copy(zrows.at[pl.ds(s * RPT, RPT)], accum.at[pl.ds(s * RPT, RPT)])

    pltpu.sync_copy(wid2d, widx_v)

    @pl.when(c == 0)
    def _():
        for k in range(5):
            _gather_rows(m1t, m1n_o, widx_v.at[s * 5 + k], rows_v, sem,
                         s * 5 + k)

    @pl.when(c == 1)
    def _():
        for k in range(5):
            _gather_rows(rmt, rmn_o, widx_v.at[s * 5 + k], rows_v, sem,
                         s * 5 + k)

    plsc.subcore_barrier()

    @pl.when(c == 0)
    def _():
        _seg_accum(src2, dst2, m1n_o, accum, src_v, dst_v, rows_v, sems,
                   s * (EROWS // 16), EROWS // 128)

    @pl.when(c == 1)
    def _():
        _seg_accum(src2, dst2, rmn_o, accum, src_v, dst_v, rows_v, sems,
                   s * (EROWS // 16), EROWS // 128)

    plsc.subcore_barrier()

    @pl.when(c == 0)
    def _():
        pltpu.sync_copy(accum.at[pl.ds(s * RPT, RPT)],
                        s2_o.at[pl.ds(s * RPT, RPT)])

    @pl.when(c == 1)
    def _():
        pltpu.sync_copy(accum.at[pl.ds(s * RPT, RPT)],
                        a2_o.at[pl.ds(s * RPT, RPT)])

    for tbl, outp in ((ewz1, xwz1_o), (ewh1, xwh1_o), (ewu1, xwu1_o)):
        for i in range(3):
            j = gw + 32 * i

            @pl.when(j < NP // 128)
            def _(tbl=tbl, outp=outp, j=j):
                _gather_rows(tbl, outp, widx_v.at[j], rows_v, sem, j)


@functools.cache
def _get_sc1():
    return pl.kernel(
        _sc1_body,
        out_type=[
            jax.ShapeDtypeStruct((NP, H), _f32),
            jax.ShapeDtypeStruct((NP, H), _f32),
            jax.ShapeDtypeStruct((NP, H), _f32),
            jax.ShapeDtypeStruct((NP, H), _f32),
            jax.ShapeDtypeStruct((NP, H), _f32),
            jax.ShapeDtypeStruct((NP, H), _f32),
            jax.ShapeDtypeStruct((NP, H), _f32),
        ],
        mesh=plsc.VectorSubcoreMesh(core_axis_name="c", subcore_axis_name="s",
                                    num_cores=2, num_subcores=16),
        scratch_types=[
            pltpu.VMEM((NP // 128, 128), jnp.int32),
            pltpu.VMEM((8, 128), jnp.int32),
            pltpu.VMEM((8, 128), jnp.int32),
            pltpu.VMEM((256, H), _f32),
            pltpu.VMEM_SHARED((NA, H), _f32),
            pltpu.SemaphoreType.DMA,
            pltpu.SemaphoreType.DMA,
        ],
    )


def _m2_body(xwz1, xwh1, s2, a2, wz2, wh2, bz, bh, m2_o):
    z2 = jax.nn.sigmoid(xwz1[...] + jnp.dot(s2[...], wz2[...],
                                            preferred_element_type=_f32)
                        + bz[...])
    p2 = jnp.tanh(xwh1[...] + jnp.dot(a2[...], wh2[...],
                                      preferred_element_type=_f32)
                  + bh[...])
    m2_o[...] = (1.0 - z2) * s2[...] + z2 * p2


_m2_tc = pl.pallas_call(
    _m2_body,
    grid=(5,),
    in_specs=[
        pl.BlockSpec((2048, H), lambda i: (i, 0)),
        pl.BlockSpec((2048, H), lambda i: (i, 0)),
        pl.BlockSpec((2048, H), lambda i: (i, 0)),
        pl.BlockSpec((2048, H), lambda i: (i, 0)),
        pl.BlockSpec((H, H), lambda i: (0, 0)),
        pl.BlockSpec((H, H), lambda i: (0, 0)),
        pl.BlockSpec((1, H), lambda i: (0, 0)),
        pl.BlockSpec((1, H), lambda i: (0, 0)),
    ],
    out_specs=pl.BlockSpec((2048, H), lambda i: (i, 0)),
    out_shape=jax.ShapeDtypeStruct((NP, H), _f32),
)


def _sc2_body(src2, dst2, m2, zrows, h0_o, h1_o,
              src_v, dst_v, rows_v, accum, semA, semB):
    sems = (semA, semB)
    c = lax.axis_index("c")
    s = lax.axis_index("s")
    gw = s * 2 + c

    pltpu.sync_copy(zrows.at[pl.ds(s * RPT, RPT)], accum.at[pl.ds(s * RPT, RPT)])
    plsc.subcore_barrier()

    _seg_accum(src2, dst2, m2, accum, src_v, dst_v, rows_v, sems,
               gw * (EROWS // 32), EROWS // 256)
    plsc.subcore_barrier()

    @pl.when(c == 0)
    def _():
        pltpu.sync_copy(accum.at[pl.ds(s * RPT, RPT)],
                        h0_o.at[pl.ds(s * RPT, RPT)])

    @pl.when(c == 1)
    def _():
        pltpu.sync_copy(accum.at[pl.ds(s * RPT, RPT)],
                        h1_o.at[pl.ds(s * RPT, RPT)])


@functools.cache
def _get_sc2():
    return pl.kernel(
        _sc2_body,
        out_type=[
            jax.ShapeDtypeStruct((NP, H), _f32),
            jax.ShapeDtypeStruct((NP, H), _f32),
        ],
        mesh=plsc.VectorSubcoreMesh(core_axis_name="c", subcore_axis_name="s",
                                    num_cores=2, num_subcores=16),
        scratch_types=[
            pltpu.VMEM((8, 128), jnp.int32),
            pltpu.VMEM((8, 128), jnp.int32),
            pltpu.VMEM((256, H), _f32),
            pltpu.VMEM_SHARED((NA, H), _f32),
            pltpu.SemaphoreType.DMA,
            pltpu.SemaphoreType.DMA,
        ],
    )


_R = 400
_VO = 896


def _head_body(hp0, hp1, xwu1, tv, qt, ptg, ww1, ww2, bw, wo, bo,
               wu2, wu3, bu, us, bs, qloss_o, ploss_o, qacc_o, pacc_o):
    step = pl.program_id(0)
    h = hp0[...] + hp1[...]
    act1 = jax.nn.relu(jnp.dot(h, ww1[...], preferred_element_type=_f32)
                       + jnp.dot(tv[...], ww2[...], preferred_element_type=_f32)
                       + bw[...])
    q = jnp.dot(act1, wo[...], preferred_element_type=_f32) + bo[...]
    mx = jnp.max(q, axis=1, keepdims=True)
    lse = jnp.log(jnp.sum(jnp.exp(q - mx), axis=1, keepdims=True)) + mx
    qtv = qt[...]
    iot = lax.broadcasted_iota(jnp.int32, (_R, _VO), 1)
    sel = jnp.sum(jnp.where(iot == qtv, q, 0.0), axis=1, keepdims=True)
    qloss_part = jnp.sum(lse - sel, keepdims=True)
    amax = jnp.min(jnp.where(q == mx, iot, _VO), axis=1, keepdims=True)
    qacc_part = jnp.sum((amax == qtv).astype(_f32), keepdims=True)

    act2 = jax.nn.relu(xwu1[...] + jnp.dot(h, wu2[...],
                                           preferred_element_type=_f32)
                       + jnp.dot(tv[...], wu3[...], preferred_element_type=_f32)
                       + bu[...])
    p = jnp.sum(act2 * us[...], axis=1, keepdims=True) + bs[...]
    ptf = ptg[...].astype(_f32)
    ploss_part = jnp.sum(jnp.maximum(p, 0.0) - p * ptf
                         + jnp.log1p(jnp.exp(-jnp.abs(p))), keepdims=True)
    pacc_part = jnp.sum(((p > 0.0).astype(jnp.int32) == ptg[...]).astype(_f32),
                        keepdims=True)

    @pl.when(step == 0)
    def _():
        zero = jnp.zeros((1, 1), _f32)
        qloss_o[...] = zero
        ploss_o[...] = zero
        qacc_o[...] = zero
        pacc_o[...] = zero

    qloss_o[...] += qloss_part
    ploss_o[...] += ploss_part
    qacc_o[...] += qacc_part
    pacc_o[...] += pacc_part

    @pl.when(step == (N // _R) - 1)
    def _():
        qloss_o[...] = qloss_o[...] / NT
        ploss_o[...] = ploss_o[...] / NT
        qacc_o[...] = qacc_o[...] / float(N)
        pacc_o[...] = pacc_o[...] / float(N)


_head_tc = pl.pallas_call(
    _head_body,
    grid=(N // _R,),
    in_specs=[
        pl.BlockSpec((_R, H), lambda i: (i, 0)),
        pl.BlockSpec((_R, H), lambda i: (i, 0)),
        pl.BlockSpec((_R, H), lambda i: (i, 0)),
        pl.BlockSpec((_R, H), lambda i: (i, 0)),
        pl.BlockSpec((_R, 1), lambda i: (i, 0)),
        pl.BlockSpec((_R, 1), lambda i: (i, 0)),
        pl.BlockSpec((H, H), lambda i: (0, 0)),
        pl.BlockSpec((H, H), lambda i: (0, 0)),
        pl.BlockSpec((1, H), lambda i: (0, 0)),
        pl.BlockSpec((H, _VO), lambda i: (0, 0)),
        pl.BlockSpec((1, _VO), lambda i: (0, 0)),
        pl.BlockSpec((H, H), lambda i: (0, 0)),
        pl.BlockSpec((H, H), lambda i: (0, 0)),
        pl.BlockSpec((1, H), lambda i: (0, 0)),
        pl.BlockSpec((1, H), lambda i: (0, 0)),
        pl.BlockSpec((1, 1), lambda i: (0, 0)),
    ],
    out_specs=[pl.BlockSpec((1, 1), lambda i: (0, 0)) for _ in range(4)],
    out_shape=[jax.ShapeDtypeStruct((1, 1), _f32) for _ in range(4)],
)


def kernel(wid, edge_index, tree_vec, p_targets, q_targets, emb, Wz, bz, Wr,
           Ur, bUr, Wh, bh, W_W, b_W, W_o, b_o, W_U, b_U, U_s, b_s):
    emb_p = jnp.pad(emb, ((0, VP - V), (0, 0)))
    bz2 = bz.reshape(1, H)
    bh2 = bh.reshape(1, H)
    bur2 = bUr.reshape(1, H)
    m1t, rmt, ewz1, ewh1, ewu1 = _tables_tc(
        emb_p, Wz[:H], Wh[:H], Wr, Ur, W_U[:H], bz2, bh2, bur2)

    wid_p = jnp.pad(wid, (0, NP - N)).reshape(NP // 128, 128)
    ep = EROWS * 128 - E
    src2 = jnp.pad(edge_index[0], (0, ep)).reshape(EROWS, 128)
    dst2 = jnp.pad(edge_index[1], (0, ep),
                   constant_values=DUMMY).reshape(EROWS, 128)
    zrows = jnp.zeros((NP, H), _f32)

    s2, a2, _m1n, _rmn, xwz1, xwh1, xwu1 = _get_sc1()(
        wid_p, src2, dst2, m1t, rmt, ewz1, ewh1, ewu1, zrows)

    m2 = _m2_tc(xwz1, xwh1, s2, a2, Wz[H:], Wh[H:], bz2, bh2)

    h0, h1 = _get_sc2()(src2, dst2, m2, zrows)

    tv_p = jnp.pad(tree_vec, ((0, 0), (0, H - L)))
    ww2_p = jnp.pad(W_W[H:], ((0, H - L), (0, 0)))
    wu3_p = jnp.pad(W_U[2 * H:], ((0, H - L), (0, 0)))
    wo_p = jnp.pad(W_o, ((0, 0), (0, _VO - V)))
    bo_p = jnp.pad(b_o, (0, _VO - V), constant_values=-1e30).reshape(1, _VO)
    qt2 = q_targets.reshape(N, 1)
    pt2 = p_targets.reshape(N, 1)

    qloss, ploss, qacc, pacc = _head_tc(
        h0, h1, xwu1[:N], tv_p, qt2, pt2,
        W_W[:H], ww2_p, b_W.reshape(1, H), wo_p, bo_p,
        W_U[H:2 * H], wu3_p, b_U.reshape(1, H),
        U_s.reshape(1, H), b_s.reshape(1, 1))

    return (qloss[0, 0], ploss[0, 0], qacc[0, 0], pacc[0, 0])


# speedup vs baseline: 5.2934x; 1.0075x over previous
import functools

import jax
import jax.numpy as jnp
from jax import lax
from jax.experimental import pallas as pl
from jax.experimental.pallas import tpu as pltpu
from jax.experimental.pallas import tpu_sc as plsc

N = 10000
E = 320000
H = 128
L = 56
V = 780
NT = 256.0

NP = 10240
VP = 784
EROWS = 2560
NA = 10112
RPT = NA // 16
DUMMY = N

_f32 = jnp.float32


def _tables_body(emb, wz1, wh1, wr, ur, wu1, bz, bh, bur,
                 m1t_o, rmt_o, ewz1_o, ewh1_o, ewu1_o):
    e = emb[...]
    ewz1 = jnp.dot(e, wz1[...], preferred_element_type=_f32)
    ewh1 = jnp.dot(e, wh1[...], preferred_element_type=_f32)
    m1 = jax.nn.sigmoid(ewz1 + bz[...]) * jnp.tanh(ewh1 + bh[...])
    rt = jax.nn.sigmoid(jnp.dot(e, wr[...], preferred_element_type=_f32)
                        + jnp.dot(m1, ur[...], preferred_element_type=_f32)
                        + bur[...])
    m1t_o[...] = m1
    rmt_o[...] = rt * m1
    ewz1_o[...] = ewz1
    ewh1_o[...] = ewh1
    ewu1_o[...] = jnp.dot(e, wu1[...], preferred_element_type=_f32)


_tables_tc = pl.pallas_call(
    _tables_body,
    out_shape=[jax.ShapeDtypeStruct((VP, H), _f32) for _ in range(5)],
)


CH = 16


def _seg_accum(src2, dst2, tbl, accum, src_c, dst_c, rows_v,
               gsems, ssems, isem, row_base, nrows):
    nch = nrows // CH
    pltpu.async_copy(src2.at[pl.ds(row_base, CH)], src_c.at[pl.ds(0, CH)],
                     isem)
    pltpu.async_copy(dst2.at[pl.ds(row_base, CH)], dst_c.at[pl.ds(0, CH)],
                     isem)

    def _buf(b):
        return rows_v.at[pl.ds(b * 128, 128)]

    def chunk_body(k, carry):
        sel = (k % 2) * CH
        sel2 = ((k + 1) % 2) * CH
        r0 = row_base + k * CH
        pltpu.make_async_copy(src2.at[pl.ds(row_base, CH)],
                              src_c.at[pl.ds(0, CH)], isem).wait()
        pltpu.make_async_copy(dst2.at[pl.ds(row_base, CH)],
                              dst_c.at[pl.ds(0, CH)], isem).wait()

        @pl.when(k + 1 < nch)
        def _():
            pltpu.async_copy(src2.at[pl.ds(r0 + CH, CH)],
                             src_c.at[pl.ds(sel2, CH)], isem)
            pltpu.async_copy(dst2.at[pl.ds(r0 + CH, CH)],
                             dst_c.at[pl.ds(sel2, CH)], isem)

        pltpu.async_copy(tbl.at[src_c.at[sel]], _buf(0), gsems[0])
        for r in range(CH):
            b = r % 2
            pltpu.make_async_copy(tbl.at[src_c.at[sel + r]], _buf(b),
                                  gsems[b]).wait()
            pltpu.async_copy(_buf(b), accum.at[dst_c.at[sel + r]], ssems[b],
                             add=True)
            if r + 1 < CH:
                b2 = (r + 1) % 2
                if r >= 1:
                    pltpu.make_async_copy(tbl.at[src_c.at[sel]], _buf(b2),
                                          ssems[b2]).wait()
                pltpu.async_copy(tbl.at[src_c.at[sel + r + 1]], _buf(b2),
                                 gsems[b2])
        pltpu.make_async_copy(tbl.at[src_c.at[sel]], _buf(0), ssems[0]).wait()
        pltpu.make_async_copy(tbl.at[src_c.at[sel]], _buf(1), ssems[1]).wait()
        return carry

    lax.fori_loop(0, nch, chunk_body, 0)


def _gather_rows(tbl, out, widx_row, rows_v, sem, out_row):
    pltpu.async_copy(tbl.at[widx_row], rows_v.at[pl.ds(0, 128)], sem).wait()
    pltpu.sync_copy(rows_v.at[pl.ds(0, 128)], out.at[pl.ds(out_row * 128, 128)])


def _sc1_body(wid2d, src2, dst2, m1t, rmt, ewz1, ewh1, ewu1, zrows,
              s2_o, a2_o, m1n_o, rmn_o, xwz1_o, xwh1_o, xwu1_o,
              src_c, dst_c, rows_v, accum,
              g0, g1, s0, s1, isem, semA):
    sem = semA
    gsems = (g0, g1)
    ssems = (s0, s1)
    c = lax.axis_index("c")
    s = lax.axis_index("s")
    gw = s * 2 + c

    pltpu.sync_copy(zrows.at[pl.ds(s * RPT, RPT)], accum.at[pl.ds(s * RPT, RPT)])

    @pl.when(s < NP // 128 // 8)
    def _():
        pltpu.sync_copy(wid2d.at[pl.ds(s * 8, 8)], dst_c.at[pl.ds(0, 8)])

        @pl.when(c == 0)
        def _():
            for k in range(8):
                _gather_rows(m1t, m1n_o, dst_c.at[k], rows_v, sem, s * 8 + k)

        @pl.when(c == 1)
        def _():
            for k in range(8):
                _gather_rows(rmt, rmn_o, dst_c.at[k], rows_v, sem, s * 8 + k)

    plsc.subcore_barrier()

    @pl.when(c == 0)
    def _():
        _seg_accum(src2, dst2, m1n_o, accum, src_c, dst_c, rows_v,
                   gsems, ssems, isem, s * (EROWS // 16), EROWS // 16)

    @pl.when(c == 1)
    def _():
        _seg_accum(src2, dst2, rmn_o, accum, src_c, dst_c, rows_v,
                   gsems, ssems, isem, s * (EROWS // 16), EROWS // 16)

    plsc.subcore_barrier()

    @pl.when(c == 0)
    def _():
        pltpu.sync_copy(accum.at[pl.ds(s * RPT, RPT)],
                        s2_o.at[pl.ds(s * RPT, RPT)])

    @pl.when(c == 1)
    def _():
        pltpu.sync_copy(accum.at[pl.ds(s * RPT, RPT)],
                        a2_o.at[pl.ds(s * RPT, RPT)])

    nsl = NP // 128 // 8
    t = gw // nsl
    sl = gw - t * nsl
    for tt, (tbl, outp) in enumerate(((ewz1, xwz1_o), (ewh1, xwh1_o),
                                      (ewu1, xwu1_o))):

        @pl.when(t == tt)
        def _(tbl=tbl, outp=outp):
            pltpu.sync_copy(wid2d.at[pl.ds(sl * 8, 8)], dst_c.at[pl.ds(0, 8)])
            for k in range(8):
                _gather_rows(tbl, outp, dst_c.at[k], rows_v, sem, sl * 8 + k)


@functools.cache
def _get_sc1():
    return pl.kernel(
        _sc1_body,
        out_type=[
            jax.ShapeDtypeStruct((NP, H), _f32),
            jax.ShapeDtypeStruct((NP, H), _f32),
            jax.ShapeDtypeStruct((NP, H), _f32),
            jax.ShapeDtypeStruct((NP, H), _f32),
            jax.ShapeDtypeStruct((NP, H), _f32),
            jax.ShapeDtypeStruct((NP, H), _f32),
            jax.ShapeDtypeStruct((NP, H), _f32),
        ],
        mesh=plsc.VectorSubcoreMesh(core_axis_name="c", subcore_axis_name="s",
                                    num_cores=2, num_subcores=16),
        scratch_types=[
            pltpu.VMEM((2 * CH, 128), jnp.int32),
            pltpu.VMEM((2 * CH, 128), jnp.int32),
            pltpu.VMEM((256, H), _f32),
            pltpu.VMEM_SHARED((NA, H), _f32),
        ] + [pltpu.SemaphoreType.DMA] * 6,
    )


def _m2_body(xwz1, xwh1, s2, a2, wz2, wh2, bz, bh, m2_o):
    z2 = jax.nn.sigmoid(xwz1[...] + jnp.dot(s2[...], wz2[...],
                                            preferred_element_type=_f32)
                        + bz[...])
    p2 = jnp.tanh(xwh1[...] + jnp.dot(a2[...], wh2[...],
                                      preferred_element_type=_f32)
                  + bh[...])
    m2_o[...] = (1.0 - z2) * s2[...] + z2 * p2


_m2_tc = pl.pallas_call(
    _m2_body,
    grid=(5,),
    in_specs=[
        pl.BlockSpec((2048, H), lambda i: (i, 0)),
        pl.BlockSpec((2048, H), lambda i: (i, 0)),
        pl.BlockSpec((2048, H), lambda i: (i, 0)),
        pl.BlockSpec((2048, H), lambda i: (i, 0)),
        pl.BlockSpec((H, H), lambda i: (0, 0)),
        pl.BlockSpec((H, H), lambda i: (0, 0)),
        pl.BlockSpec((1, H), lambda i: (0, 0)),
        pl.BlockSpec((1, H), lambda i: (0, 0)),
    ],
    out_specs=pl.BlockSpec((2048, H), lambda i: (i, 0)),
    out_shape=jax.ShapeDtypeStruct((NP, H), _f32),
)


def _sc2_body(src2, dst2, m2, zrows, h0_o, h1_o,
              src_c, dst_c, rows_v, accum,
              g0, g1, s0, s1, isem):
    gsems = (g0, g1)
    ssems = (s0, s1)
    c = lax.axis_index("c")
    s = lax.axis_index("s")
    gw = s * 2 + c

    pltpu.sync_copy(zrows.at[pl.ds(s * RPT, RPT)], accum.at[pl.ds(s * RPT, RPT)])
    plsc.subcore_barrier()

    _seg_accum(src2, dst2, m2, accum, src_c, dst_c, rows_v,
               gsems, ssems, isem, gw * (EROWS // 32), EROWS // 32)
    plsc.subcore_barrier()

    @pl.when(c == 0)
    def _():
        pltpu.sync_copy(accum.at[pl.ds(s * RPT, RPT)],
                        h0_o.at[pl.ds(s * RPT, RPT)])

    @pl.when(c == 1)
    def _():
        pltpu.sync_copy(accum.at[pl.ds(s * RPT, RPT)],
                        h1_o.at[pl.ds(s * RPT, RPT)])


@functools.cache
def _get_sc2():
    return pl.kernel(
        _sc2_body,
        out_type=[
            jax.ShapeDtypeStruct((NP, H), _f32),
            jax.ShapeDtypeStruct((NP, H), _f32),
        ],
        mesh=plsc.VectorSubcoreMesh(core_axis_name="c", subcore_axis_name="s",
                                    num_cores=2, num_subcores=16),
        scratch_types=[
            pltpu.VMEM((2 * CH, 128), jnp.int32),
            pltpu.VMEM((2 * CH, 128), jnp.int32),
            pltpu.VMEM((256, H), _f32),
            pltpu.VMEM_SHARED((NA, H), _f32),
        ] + [pltpu.SemaphoreType.DMA] * 5,
    )


_R = 400
_VO = 896


def _head_body(hp0, hp1, xwu1, tv, qt, ptg, ww1, ww2, bw, wo, bo,
               wu2, wu3, bu, us, bs, qloss_o, ploss_o, qacc_o, pacc_o):
    step = pl.program_id(0)
    h = hp0[...] + hp1[...]
    act1 = jax.nn.relu(jnp.dot(h, ww1[...], preferred_element_type=_f32)
                       + jnp.dot(tv[...], ww2[...], preferred_element_type=_f32)
                       + bw[...])
    q = jnp.dot(act1, wo[...], preferred_element_type=_f32) + bo[...]
    mx = jnp.max(q, axis=1, keepdims=True)
    lse = jnp.log(jnp.sum(jnp.exp(q - mx), axis=1, keepdims=True)) + mx
    qtv = qt[...]
    iot = lax.broadcasted_iota(jnp.int32, (_R, _VO), 1)
    sel = jnp.sum(jnp.where(iot == qtv, q, 0.0), axis=1, keepdims=True)
    qloss_part = jnp.sum(lse - sel, keepdims=True)
    amax = jnp.min(jnp.where(q == mx, iot, _VO), axis=1, keepdims=True)
    qacc_part = jnp.sum((amax == qtv).astype(_f32), keepdims=True)

    act2 = jax.nn.relu(xwu1[...] + jnp.dot(h, wu2[...],
                                           preferred_element_type=_f32)
                       + jnp.dot(tv[...], wu3[...], preferred_element_type=_f32)
                       + bu[...])
    p = jnp.sum(act2 * us[...], axis=1, keepdims=True) + bs[...]
    ptf = ptg[...].astype(_f32)
    ploss_part = jnp.sum(jnp.maximum(p, 0.0) - p * ptf
                         + jnp.log1p(jnp.exp(-jnp.abs(p))), keepdims=True)
    pacc_part = jnp.sum(((p > 0.0).astype(jnp.int32) == ptg[...]).astype(_f32),
                        keepdims=True)

    @pl.when(step == 0)
    def _():
        zero = jnp.zeros((1, 1), _f32)
        qloss_o[...] = zero
        ploss_o[...] = zero
        qacc_o[...] = zero
        pacc_o[...] = zero

    qloss_o[...] += qloss_part
    ploss_o[...] += ploss_part
    qacc_o[...] += qacc_part
    pacc_o[...] += pacc_part

    @pl.when(step == (N // _R) - 1)
    def _():
        qloss_o[...] = qloss_o[...] / NT
        ploss_o[...] = ploss_o[...] / NT
        qacc_o[...] = qacc_o[...] / float(N)
        pacc_o[...] = pacc_o[...] / float(N)


_head_tc = pl.pallas_call(
    _head_body,
    grid=(N // _R,),
    in_specs=[
        pl.BlockSpec((_R, H), lambda i: (i, 0)),
        pl.BlockSpec((_R, H), lambda i: (i, 0)),
        pl.BlockSpec((_R, H), lambda i: (i, 0)),
        pl.BlockSpec((_R, H), lambda i: (i, 0)),
        pl.BlockSpec((_R, 1), lambda i: (i, 0)),
        pl.BlockSpec((_R, 1), lambda i: (i, 0)),
        pl.BlockSpec((H, H), lambda i: (0, 0)),
        pl.BlockSpec((H, H), lambda i: (0, 0)),
        pl.BlockSpec((1, H), lambda i: (0, 0)),
        pl.BlockSpec((H, _VO), lambda i: (0, 0)),
        pl.BlockSpec((1, _VO), lambda i: (0, 0)),
        pl.BlockSpec((H, H), lambda i: (0, 0)),
        pl.BlockSpec((H, H), lambda i: (0, 0)),
        pl.BlockSpec((1, H), lambda i: (0, 0)),
        pl.BlockSpec((1, H), lambda i: (0, 0)),
        pl.BlockSpec((1, 1), lambda i: (0, 0)),
    ],
    out_specs=[pl.BlockSpec((1, 1), lambda i: (0, 0)) for _ in range(4)],
    out_shape=[jax.ShapeDtypeStruct((1, 1), _f32) for _ in range(4)],
)


def kernel(wid, edge_index, tree_vec, p_targets, q_targets, emb, Wz, bz, Wr,
           Ur, bUr, Wh, bh, W_W, b_W, W_o, b_o, W_U, b_U, U_s, b_s):
    emb_p = jnp.pad(emb, ((0, VP - V), (0, 0)))
    bz2 = bz.reshape(1, H)
    bh2 = bh.reshape(1, H)
    bur2 = bUr.reshape(1, H)
    m1t, rmt, ewz1, ewh1, ewu1 = _tables_tc(
        emb_p, Wz[:H], Wh[:H], Wr, Ur, W_U[:H], bz2, bh2, bur2)

    wid_p = jnp.pad(wid, (0, 96 * 128 - N)).reshape(96, 128)
    ep = EROWS * 128 - E
    src2 = jnp.pad(edge_index[0], (0, ep)).reshape(EROWS, 128)
    dst2 = jnp.pad(edge_index[1], (0, ep),
                   constant_values=DUMMY).reshape(EROWS, 128)
    zrows = jnp.zeros((NP, H), _f32)

    s2, a2, _m1n, _rmn, xwz1, xwh1, xwu1 = _get_sc1()(
        wid_p, src2, dst2, m1t, rmt, ewz1, ewh1, ewu1, zrows)

    m2 = _m2_tc(xwz1, xwh1, s2, a2, Wz[H:], Wh[H:], bz2, bh2)

    h0, h1 = _get_sc2()(src2, dst2, m2, zrows)

    tv_p = jnp.pad(tree_vec, ((0, 0), (0, H - L)))
    ww2_p = jnp.pad(W_W[H:], ((0, H - L), (0, 0)))
    wu3_p = jnp.pad(W_U[2 * H:], ((0, H - L), (0, 0)))
    wo_p = jnp.pad(W_o, ((0, 0), (0, _VO - V)))
    bo_p = jnp.pad(b_o, (0, _VO - V), constant_values=-1e30).reshape(1, _VO)
    qt2 = q_targets.reshape(N, 1)
    pt2 = p_targets.reshape(N, 1)

    qloss, ploss, qacc, pacc = _head_tc(
        h0, h1, xwu1[:N], tv_p, qt2, pt2,
        W_W[:H], ww2_p, b_W.reshape(1, H), wo_p, bo_p,
        W_U[H:2 * H], wu3_p, b_U.reshape(1, H),
        U_s.reshape(1, H), b_s.reshape(1, 1))

    return (qloss[0, 0], ploss[0, 0], qacc[0, 0], pacc[0, 0])
